# Initial kernel scaffold; baseline (speedup 1.0000x reference)
import jax
import jax.numpy as jnp
from jax.experimental import pallas as pl


def kernel(masked_X, masked_E, We, be, W1, b1, W2, b2, gamma, beta, W3, b3, W4, b4):
    raise NotImplementedError("write your pallas kernel here")


# speedup vs baseline: 30.2064x; 30.2064x over previous
import jax
import jax.numpy as jnp
from jax.experimental import pallas as pl

_B, _N, _D, _De, _H = 16, 128, 128, 16, 128
_M = _B * _N
_GI = 16
_IC = _N // _GI


def _prep_body(x_ref, xu_ref, mf_ref):
    x = x_ref[...]
    rs = jnp.sum(x, axis=1, keepdims=True)
    m = (rs != 0.0).astype(jnp.float32)
    val = x * m
    has = m
    s = 1
    while s < _M:
        val_sh = jnp.concatenate(
            [jnp.zeros((s, _D), jnp.float32), val[:-s]], axis=0)
        has_sh = jnp.concatenate(
            [jnp.zeros((s, 1), jnp.float32), has[:-s]], axis=0)
        val = jnp.where(has > 0, val, val_sh)
        has = jnp.maximum(has, has_sh)
        s *= 2
    xu_ref[...] = val
    mf_ref[...] = m


def _heavy_body(e_ref, xu_ref, we_ref, be_ref, a_ref):
    ic = pl.program_id(1)
    e = e_ref[...].reshape(_GI * _N, _De)
    emb = jax.lax.dot_general(
        e, we_ref[...], (((1,), (0,)), ((), ())),
        preferred_element_type=jnp.float32) + be_ref[...]
    rs = jnp.sum(e, axis=1, keepdims=True)
    mv = (rs != 0.0).astype(jnp.float32)
    xu = xu_ref[...].reshape(_GI, _D)
    emb3 = emb.reshape(_GI, _N, _D)
    msg = jnp.maximum(emb3 + xu[:, None, :], 0.0) * mv.reshape(_GI, _N, 1)
    contrib = jnp.sum(msg, axis=0)

    @pl.when(ic == 0)
    def _init():
        a_ref[0] = contrib

    @pl.when(ic != 0)
    def _acc():
        a_ref[0] += contrib


def _post_body(a_ref, x_ref, mf_ref, w1_ref, b1_ref, w2_ref, b2_ref,
               g_ref, bt_ref, w3_ref, b3_ref, w4_ref, b4_ref, o_ref):
    a = a_ref[...]
    m = mf_ref[...]
    r = jnp.concatenate([m[1:], jnp.ones((1, 1), jnp.float32)], axis=0)
    v = a
    s = 1
    while s < _M:
        v_sh = jnp.concatenate(
            [v[s:], jnp.zeros((s, _D), jnp.float32)], axis=0)
        r_sh = jnp.concatenate(
            [r[s:], jnp.ones((s, 1), jnp.float32)], axis=0)
        v = v + jnp.where(r > 0, 0.0, v_sh)
        r = jnp.maximum(r, r_sh)
        s *= 2
    h = x_ref[...] + v
    h = jnp.maximum(jnp.dot(h, w1_ref[...],
                            preferred_element_type=jnp.float32) + b1_ref[...],
                    0.0)
    h = jnp.dot(h, w2_ref[...],
                preferred_element_type=jnp.float32) + b2_ref[...]
    hm = h * m
    sums = jnp.sum(hm.reshape(_B, _N, _H), axis=1)
    counts = jnp.sum(m.reshape(_B, _N, 1), axis=1)
    pooled = sums / jnp.maximum(counts, 1.0)
    mu = jnp.mean(pooled, axis=1, keepdims=True)
    var = jnp.mean((pooled - mu) ** 2, axis=1, keepdims=True)
    normed = (pooled - mu) / jnp.sqrt(var + 1e-5) * g_ref[...] + bt_ref[...]
    z = jnp.maximum(jnp.dot(normed, w3_ref[...],
                            preferred_element_type=jnp.float32) + b3_ref[...],
                    0.0)
    z = jnp.dot(z, w4_ref[...],
                preferred_element_type=jnp.float32) + b4_ref[...]
    o_ref[...] = jax.nn.sigmoid(z)


def kernel(masked_X, masked_E, We, be, W1, b1, W2, b2, gamma, beta,
           W3, b3, W4, b4):
    f32 = jnp.float32
    Xf = masked_X.reshape(_M, _D)
    xu, mf = pl.pallas_call(
        _prep_body,
        out_shape=[jax.ShapeDtypeStruct((_M, _D), f32),
                   jax.ShapeDtypeStruct((_M, 1), f32)],
    )(Xf)

    xu4 = xu.reshape(_B, _IC, _GI, _D)
    a = pl.pallas_call(
        _heavy_body,
        grid=(_B, _IC),
        in_specs=[
            pl.BlockSpec((1, _GI, _N, _De), lambda b, ic: (b, ic, 0, 0)),
            pl.BlockSpec((1, 1, _GI, _D), lambda b, ic: (b, ic, 0, 0)),
            pl.BlockSpec((_De, _D), lambda b, ic: (0, 0)),
            pl.BlockSpec((1, _D), lambda b, ic: (0, 0)),
        ],
        out_specs=pl.BlockSpec((1, _N, _D), lambda b, ic: (b, 0, 0)),
        out_shape=jax.ShapeDtypeStruct((_B, _N, _D), f32),
    )(masked_E, xu4, We, be.reshape(1, _D))

    score = pl.pallas_call(
        _post_body,
        out_shape=jax.ShapeDtypeStruct((_B, 1), f32),
    )(a.reshape(_M, _D), Xf, mf,
      W1, b1.reshape(1, _H), W2, b2.reshape(1, _H),
      gamma.reshape(1, _H), beta.reshape(1, _H),
      W3, b3.reshape(1, _H), W4, b4.reshape(1, 1))
    return score


# speedup vs baseline: 31.7577x; 1.0514x over previous
import jax
import jax.numpy as jnp
from jax.experimental import pallas as pl

_B, _N, _D, _De, _H = 16, 128, 128, 16, 128
_M = _B * _N
_GI = 16
_IC = _N // _GI


def _prep_body(x_ref, xu_ref, mf_ref):
    x = x_ref[...]
    rs = jnp.sum(x, axis=1, keepdims=True)
    m = (rs != 0.0).astype(jnp.float32)
    val = x * m
    has = m
    s = 1
    while s < _M:
        val_sh = jnp.concatenate(
            [jnp.zeros((s, _D), jnp.float32), val[:-s]], axis=0)
        has_sh = jnp.concatenate(
            [jnp.zeros((s, 1), jnp.float32), has[:-s]], axis=0)
        val = jnp.where(has > 0, val, val_sh)
        has = jnp.maximum(has, has_sh)
        s *= 2
    xu_ref[...] = val
    mf_ref[...] = m


_NG = _N * _De // 128
_EPR = 128 // _De


def _heavy_body(e_ref, xu_ref, wbd_ref, s_ref, be_ref, a_ref):
    ic = pl.program_id(1)
    lhs = e_ref[...].reshape(_GI * _NG, 128)
    rs8 = jax.lax.dot_general(
        lhs, s_ref[...], (((1,), (0,)), ((), ())),
        preferred_element_type=jnp.float32)
    xu = xu_ref[...].reshape(_GI, _D)
    xq = (jnp.concatenate([xu, xu], axis=1)
          + jnp.concatenate([be_ref[...], be_ref[...]], axis=1))
    for p in range(4):
        emb = jax.lax.dot_general(
            lhs, wbd_ref[p], (((1,), (0,)), ((), ())),
            preferred_element_type=jnp.float32)
        emb3 = emb.reshape(_GI, _NG, 256)
        msg = jnp.maximum(emb3 + xq[:, None, :], 0.0)
        m0 = (rs8[:, p:p + 1] != 0.0).astype(jnp.float32).reshape(_GI, _NG, 1)
        m1 = (rs8[:, p + 4:p + 5] != 0.0).astype(jnp.float32).reshape(
            _GI, _NG, 1)
        acc_a = jnp.sum(msg[:, :, :128] * m0, axis=0)
        acc_b = jnp.sum(msg[:, :, 128:] * m1, axis=0)

        @pl.when(ic == 0)
        def _init(p=p, acc_a=acc_a, acc_b=acc_b):
            a_ref[0, :, p, :] = acc_a
            a_ref[0, :, p + 4, :] = acc_b

        @pl.when(ic != 0)
        def _acc(p=p, acc_a=acc_a, acc_b=acc_b):
            a_ref[0, :, p, :] += acc_a
            a_ref[0, :, p + 4, :] += acc_b


def _post_body(a_ref, x_ref, mf_ref, w1_ref, b1_ref, w2_ref, b2_ref,
               g_ref, bt_ref, w3_ref, b3_ref, w4_ref, b4_ref, o_ref):
    a = a_ref[...]
    m = mf_ref[...]
    r = jnp.concatenate([m[1:], jnp.ones((1, 1), jnp.float32)], axis=0)
    v = a
    s = 1
    while s < _M:
        v_sh = jnp.concatenate(
            [v[s:], jnp.zeros((s, _D), jnp.float32)], axis=0)
        r_sh = jnp.concatenate(
            [r[s:], jnp.ones((s, 1), jnp.float32)], axis=0)
        v = v + jnp.where(r > 0, 0.0, v_sh)
        r = jnp.maximum(r, r_sh)
        s *= 2
    h = x_ref[...] + v
    h = jnp.maximum(jnp.dot(h, w1_ref[...],
                            preferred_element_type=jnp.float32) + b1_ref[...],
                    0.0)
    h = jnp.dot(h, w2_ref[...],
                preferred_element_type=jnp.float32) + b2_ref[...]
    hm = h * m
    sums = jnp.sum(hm.reshape(_B, _N, _H), axis=1)
    counts = jnp.sum(m.reshape(_B, _N, 1), axis=1)
    pooled = sums / jnp.maximum(counts, 1.0)
    mu = jnp.mean(pooled, axis=1, keepdims=True)
    var = jnp.mean((pooled - mu) ** 2, axis=1, keepdims=True)
    normed = (pooled - mu) / jnp.sqrt(var + 1e-5) * g_ref[...] + bt_ref[...]
    z = jnp.maximum(jnp.dot(normed, w3_ref[...],
                            preferred_element_type=jnp.float32) + b3_ref[...],
                    0.0)
    z = jnp.dot(z, w4_ref[...],
                preferred_element_type=jnp.float32) + b4_ref[...]
    o_ref[...] = jax.nn.sigmoid(z)


def kernel(masked_X, masked_E, We, be, W1, b1, W2, b2, gamma, beta,
           W3, b3, W4, b4):
    f32 = jnp.float32
    Xf = masked_X.reshape(_M, _D)
    xu, mf = pl.pallas_call(
        _prep_body,
        out_shape=[jax.ShapeDtypeStruct((_M, _D), f32),
                   jax.ShapeDtypeStruct((_M, 1), f32)],
    )(Xf)

    xu4 = xu.reshape(_B, _IC, _GI, _D)
    e4 = masked_E.reshape(_B, _N, _NG, 128)
    wbd = jnp.zeros((4, 128, 256), f32)
    for p in range(4):
        wbd = wbd.at[p, _De * p:_De * (p + 1), 0:_D].set(We)
        wbd = wbd.at[p, _De * (p + 4):_De * (p + 5), _D:2 * _D].set(We)
    smat = (jnp.arange(128)[:, None] // _De ==
            jnp.arange(_EPR)[None, :]).astype(f32)
    a = pl.pallas_call(
        _heavy_body,
        grid=(_B, _IC),
        in_specs=[
            pl.BlockSpec((1, _GI, _NG, 128), lambda b, ic: (b, ic, 0, 0)),
            pl.BlockSpec((1, 1, _GI, _D), lambda b, ic: (b, ic, 0, 0)),
            pl.BlockSpec((4, 128, 256), lambda b, ic: (0, 0, 0)),
            pl.BlockSpec((128, _EPR), lambda b, ic: (0, 0)),
            pl.BlockSpec((1, _D), lambda b, ic: (0, 0)),
        ],
        out_specs=pl.BlockSpec((1, _NG, _EPR, _D), lambda b, ic: (b, 0, 0, 0)),
        out_shape=jax.ShapeDtypeStruct((_B, _NG, _EPR, _D), f32),
    )(e4, xu4, wbd, smat, be.reshape(1, _D))

    score = pl.pallas_call(
        _post_body,
        out_shape=jax.ShapeDtypeStruct((_B, 1), f32),
    )(a.reshape(_M, _D), Xf, mf,
      W1, b1.reshape(1, _H), W2, b2.reshape(1, _H),
      gamma.reshape(1, _H), beta.reshape(1, _H),
      W3, b3.reshape(1, _H), W4, b4.reshape(1, 1))
    return score


# speedup vs baseline: 62.7985x; 1.9774x over previous
import jax
import jax.numpy as jnp
from jax.experimental import pallas as pl
from jax.experimental.pallas import tpu as pltpu

_B, _N, _D, _De, _H = 16, 128, 128, 16, 128
_M = _B * _N
_NG = _N * _De // 128
_EPR = 128 // _De


def _body(e_ref, x_ref, wbd_ref, s_ref, be_ref,
          w1_ref, b1_ref, w2_ref, b2_ref, g_ref, bt_ref,
          w3_ref, b3_ref, w4_ref, b4_ref,
          o_ref, xu_s, mf_s, a_s):
    b = pl.program_id(0)
    f32 = jnp.float32

    @pl.when(b == 0)
    def _prep():
        x = x_ref[...]
        rs = jnp.sum(x, axis=1, keepdims=True)
        m = (rs != 0.0).astype(f32)
        val = x * m
        has = m
        s = 1
        while s < _M:
            val_sh = jnp.concatenate(
                [jnp.zeros((s, _D), f32), val[:-s]], axis=0)
            has_sh = jnp.concatenate(
                [jnp.zeros((s, 1), f32), has[:-s]], axis=0)
            val = jnp.where(has > 0, val, val_sh)
            has = jnp.maximum(has, has_sh)
            s *= 2
        xu_s[...] = val
        mf_s[...] = m

    lhs = e_ref[...].reshape(_N * _NG, 128)
    rs8 = jax.lax.dot_general(
        lhs, s_ref[...], (((1,), (0,)), ((), ())),
        preferred_element_type=f32)
    xu = xu_s[pl.ds(b * _N, _N), :]
    xq = (jnp.concatenate([xu, xu], axis=1)
          + jnp.concatenate([be_ref[...], be_ref[...]], axis=1))
    for p in range(4):
        emb = jax.lax.dot_general(
            lhs, wbd_ref[p], (((1,), (0,)), ((), ())),
            preferred_element_type=f32)
        emb3 = emb.reshape(_N, _NG, 256)
        msg = jnp.maximum(emb3 + xq[:, None, :], 0.0)
        m0 = (rs8[:, p:p + 1] != 0.0).astype(f32).reshape(_N, _NG, 1)
        m1 = (rs8[:, p + 4:p + 5] != 0.0).astype(f32).reshape(_N, _NG, 1)
        a_s[pl.ds(b * _NG, _NG), p * _D:(p + 1) * _D] = (
            jnp.sum(msg[:, :, :128] * m0, axis=0))
        a_s[pl.ds(b * _NG, _NG), (p + 4) * _D:(p + 5) * _D] = (
            jnp.sum(msg[:, :, 128:] * m1, axis=0))

    @pl.when(b == _B - 1)
    def _post():
        af = a_s[...].reshape(_B * _NG, _EPR, _D).reshape(_M, _D)
        m = mf_s[...]
        r = jnp.concatenate([m[1:], jnp.ones((1, 1), f32)], axis=0)
        v = af
        s = 1
        while s < _M:
            v_sh = jnp.concatenate(
                [v[s:], jnp.zeros((s, _D), f32)], axis=0)
            r_sh = jnp.concatenate(
                [r[s:], jnp.ones((s, 1), f32)], axis=0)
            v = v + jnp.where(r > 0, 0.0, v_sh)
            r = jnp.maximum(r, r_sh)
            s *= 2
        h = x_ref[...] + v
        h = jnp.maximum(jnp.dot(h, w1_ref[...],
                                preferred_element_type=f32) + b1_ref[...],
                        0.0)
        h = jnp.dot(h, w2_ref[...],
                    preferred_element_type=f32) + b2_ref[...]
        hm = h * m
        sums = jnp.sum(hm.reshape(_B, _N, _H), axis=1)
        counts = jnp.sum(m.reshape(_B, _N, 1), axis=1)
        pooled = sums / jnp.maximum(counts, 1.0)
        mu = jnp.mean(pooled, axis=1, keepdims=True)
        var = jnp.mean((pooled - mu) ** 2, axis=1, keepdims=True)
        normed = ((pooled - mu) / jnp.sqrt(var + 1e-5) * g_ref[...]
                  + bt_ref[...])
        z = jnp.maximum(jnp.dot(normed, w3_ref[...],
                                preferred_element_type=f32) + b3_ref[...],
                        0.0)
        z = jnp.dot(z, w4_ref[...],
                    preferred_element_type=f32) + b4_ref[...]
        o_ref[...] = jax.nn.sigmoid(z)


def kernel(masked_X, masked_E, We, be, W1, b1, W2, b2, gamma, beta,
           W3, b3, W4, b4):
    f32 = jnp.float32
    Xf = masked_X.reshape(_M, _D)
    e4 = masked_E.reshape(_B, _N, _NG, 128)
    wbd = jnp.zeros((4, 128, 256), f32)
    for p in range(4):
        wbd = wbd.at[p, _De * p:_De * (p + 1), 0:_D].set(We)
        wbd = wbd.at[p, _De * (p + 4):_De * (p + 5), _D:2 * _D].set(We)
    smat = (jnp.arange(128)[:, None] // _De ==
            jnp.arange(_EPR)[None, :]).astype(f32)
    cmap2 = lambda b: (0, 0)
    cmap3 = lambda b: (0, 0, 0)
    score = pl.pallas_call(
        _body,
        grid=(_B,),
        in_specs=[
            pl.BlockSpec((1, _N, _NG, 128), lambda b: (b, 0, 0, 0)),
            pl.BlockSpec((_M, _D), cmap2),
            pl.BlockSpec((4, 128, 256), cmap3),
            pl.BlockSpec((128, _EPR), cmap2),
            pl.BlockSpec((1, _D), cmap2),
            pl.BlockSpec((_D, _H), cmap2),
            pl.BlockSpec((1, _H), cmap2),
            pl.BlockSpec((_H, _H), cmap2),
            pl.BlockSpec((1, _H), cmap2),
            pl.BlockSpec((1, _H), cmap2),
            pl.BlockSpec((1, _H), cmap2),
            pl.BlockSpec((_H, _H), cmap2),
            pl.BlockSpec((1, _H), cmap2),
            pl.BlockSpec((_H, 1), cmap2),
            pl.BlockSpec((1, 1), cmap2),
        ],
        out_specs=pl.BlockSpec((_B, 1), cmap2),
        out_shape=jax.ShapeDtypeStruct((_B, 1), f32),
        scratch_shapes=[
            pltpu.VMEM((_M, _D), f32),
            pltpu.VMEM((_M, 1), f32),
            pltpu.VMEM((_B * _NG, _EPR * _D), f32),
        ],
    )(e4, Xf, wbd, smat, be.reshape(1, _D),
      W1, b1.reshape(1, _H), W2, b2.reshape(1, _H),
      gamma.reshape(1, _H), beta.reshape(1, _H),
      W3, b3.reshape(1, _H), W4, b4.reshape(1, 1))
    return score
